# baseline (speedup 1.0000x reference)
import numpy as np
import jax
import jax.numpy as jnp
from jax.experimental import pallas as pl
from jax.experimental.pallas import tpu as pltpu

_B, _L, _C, _D, _PRED, _NH, _DFF, _NL, _BOT = 4, 512, 7, 128, 96, 8, 256, 2, 16
_DH = _D // _NH

f32 = jnp.float32
bf16 = jnp.bfloat16


def _posemb_np():
    pos = np.arange(_L)[:, None].astype(np.float32)
    div = np.exp(np.arange(0, _D, 2).astype(np.float32) * (-np.log(10000.0) / _D))
    pe = np.zeros((_L, _D), np.float32)
    pe[:, 0::2] = np.sin(pos * div)
    pe[:, 1::2] = np.cos(pos * div)
    return pe


_PE = _posemb_np()


def _ln(x, g, b):
    m = jnp.mean(x, axis=1, keepdims=True)
    v = jnp.mean((x - m) ** 2, axis=1, keepdims=True)
    return (x - m) / jnp.sqrt(v + 1e-5) * g + b


def _bdot(a, b):
    return jax.lax.dot_general(a.astype(bf16), b.astype(bf16),
                               (((1,), (0,)), ((), ())),
                               preferred_element_type=f32)


def _dot(a, b):
    return jax.lax.dot_general(a, b, (((1,), (0,)), ((), ())),
                               preferred_element_type=f32)


def _dot_t(a, b):
    return jax.lax.dot_general(a, b, (((1,), (1,)), ((), ())),
                               preferred_element_type=f32)


def _embed_kernel(xr_ref, ve1w_ref, ve1b_ref, ve2w_ref, ve2b_ref,
                  h_ref, mu_ref, sd_ref):
    x = xr_ref[...]
    m = jnp.mean(x, axis=1, keepdims=True)
    xc = x - m
    m2 = jnp.mean(xc, axis=1, keepdims=True)
    var = jnp.mean((xc - m2) ** 2, axis=1, keepdims=True)
    std = jnp.sqrt(var + 1e-5)
    xn = xc / std
    v = _bdot(xn, ve1w_ref[...]) + ve1b_ref[...]
    v16 = v.astype(bf16).astype(f32)
    h_ref[...] = _bdot(v16, ve2w_ref[...]) + ve2b_ref[...]
    mu_ref[...] = m
    sd_ref[...] = std


def _row_kernel(*refs):
    (h_ref, rpw_ref, rpb_ref, pe_ref) = refs[:4]
    layer_refs = refs[4:4 + 16 * _NL]
    (bng_ref, bnb_ref, bnrm_ref, bnrv_ref, oh1w_ref, oh1b_ref) = refs[4 + 16 * _NL:4 + 16 * _NL + 6]
    (t_ref, pb_ref) = refs[4 + 16 * _NL + 6:]

    h = h_ref[0]
    h16 = h.astype(bf16)

    resid = _dot(h16, rpw_ref[...]) + rpb_ref[...]

    hq = h16.astype(f32)
    nrm = jnp.sqrt(jnp.sum(hq * hq, axis=1, keepdims=True))
    dots = jnp.sum(hq[:_L - 1] * hq[1:], axis=1, keepdims=True)
    cos = dots / (nrm[:_L - 1] * nrm[1:] + 1e-12)
    pbt = jnp.clip((1.0 - cos) * 0.5, 0.0, 1.0)
    pb = jnp.concatenate([jnp.ones((1, 1), f32), pbt], axis=0)

    isub = jax.lax.broadcasted_iota(jnp.int32, (_L, 1), 0)
    bmaskf = jnp.where(pb >= 0.5, 1.0, 0.0)
    bmaskf = jnp.where(isub == 0, 1.0, bmaskf)

    ii = jax.lax.broadcasted_iota(jnp.int32, (_L, _L), 0).astype(f32)
    jj = jax.lax.broadcasted_iota(jnp.int32, (_L, _L), 1).astype(f32)
    tri = jnp.where(jj <= ii, 1.0, 0.0)
    c_s = _dot(tri, bmaskf)

    eye = jnp.where(ii == jj, 1.0, 0.0)
    ones_row = jnp.ones((1, _L), f32)
    b_l = _dot(ones_row, eye * bmaskf)
    c_l = _dot(b_l, jnp.where(ii <= jj, 1.0, 0.0))

    P = jnp.where((jnp.abs(c_l - 1.0 - ii) < 0.5) & (b_l > 0.5), 1.0, 0.0)
    G = jnp.where(jnp.abs(c_s - 1.0 - jj) < 0.5, 1.0, 0.0)

    z = _dot(P, h16) + pe_ref[...]

    inv_sqrt_dh = 1.0 / np.sqrt(float(_DH))
    for l in range(_NL):
        (wq, bq, wk, bk, wv, bv, wo, bo, c1w, c1b, c2w, c2b,
         g1, b1, g2, b2) = (r[...] for r in layer_refs[16 * l:16 * (l + 1)])
        z16 = z.astype(bf16)
        q16 = ((_dot(z16, wq) + bq) * inv_sqrt_dh).astype(bf16)
        k16 = (_dot(z16, wk) + bk).astype(bf16)
        v16 = (_dot(z16, wv) + bv).astype(bf16)
        outs = []
        for hh in range(_NH):
            sl = slice(hh * _DH, (hh + 1) * _DH)
            s = _dot_t(q16[:, sl], k16[:, sl])
            s = s - jnp.max(s, axis=1, keepdims=True)
            e = jnp.exp(s)
            r = 1.0 / jnp.sum(e, axis=1, keepdims=True)
            outs.append(_dot(e, v16[:, sl]) * r)
        o16 = jnp.concatenate(outs, axis=1).astype(bf16)
        o = _dot(o16, wo) + bo
        x1 = _ln(z + o, g1, b1)
        y = jax.nn.gelu(_dot(x1.astype(bf16), c1w) + c1b)
        y = _dot(y.astype(bf16), c2w) + c2b
        z = _ln(x1 + y, g2, b2)

    z = (z - bnrm_ref[...]) / jnp.sqrt(bnrv_ref[...] + 1e-5) * bng_ref[...] + bnb_ref[...]

    expanded = _dot(G, z.astype(bf16))

    w = jnp.clip(pb, 1e-4, 1.0)
    bb = w * expanded
    aa = 1.0 - w
    d = 1
    while d < _L:
        a_sh = jnp.concatenate([jnp.ones((d, 1), f32), aa[:_L - d]], axis=0)
        b_sh = jnp.concatenate([jnp.zeros((d, _D), f32), bb[:_L - d]], axis=0)
        bb = aa * b_sh + bb
        aa = aa * a_sh
        d *= 2
    hs = bb + resid

    t = _dot(jnp.transpose(hs.astype(bf16)), oh1w_ref[...]) + oh1b_ref[...]

    t_ref[...] = t.reshape(1, _D, _BOT)
    pb_ref[...] = pb.reshape(1, _L, 1)


def _head_kernel(t_ref, w_ref, b_ref, mu_ref, sd_ref, out_ref):
    t = t_ref[...]
    dec = _dot(t.astype(bf16), w_ref[...]) + b_ref[...]
    out_ref[...] = dec * sd_ref[...] + mu_ref[...]


def kernel(x_enc, x_mark_enc, x_dec, x_mark_dec, params):
    p = params
    bc = _B * _C

    xr = jnp.transpose(x_enc, (0, 2, 1)).reshape(bc, _L).astype(f32)

    def rv(a, n):
        return a.reshape(1, n)

    def full_spec(a):
        nd = a.ndim
        return pl.BlockSpec(a.shape, lambda i, _n=nd: (0,) * _n)

    emb_operands = [xr, p['ve1_w'], rv(p['ve1_b'], _BOT),
                    p['ve2_w'], rv(p['ve2_b'], _L * _D)]
    h_all, mu, sd = pl.pallas_call(
        _embed_kernel,
        out_shape=[jax.ShapeDtypeStruct((bc, _L * _D), f32),
                   jax.ShapeDtypeStruct((bc, 1), f32),
                   jax.ShapeDtypeStruct((bc, 1), f32)],
    )(*emb_operands)

    def wcast(a):
        return a.astype(bf16)

    operands = [
        h_all.reshape(bc, _L, _D),
        wcast(p['rp_w']), rv(p['rp_b'], _D),
        jnp.asarray(_PE),
    ]
    for lp in p['layers']:
        operands += [
            wcast(lp['wq']), rv(lp['bq'], _D), wcast(lp['wk']), rv(lp['bk'], _D),
            wcast(lp['wv']), rv(lp['bv'], _D), wcast(lp['wo']), rv(lp['bo'], _D),
            wcast(lp['c1_w']), rv(lp['c1_b'], _DFF), wcast(lp['c2_w']), rv(lp['c2_b'], _D),
            rv(lp['ln1_g'], _D), rv(lp['ln1_b'], _D),
            rv(lp['ln2_g'], _D), rv(lp['ln2_b'], _D),
        ]
    operands += [
        rv(p['bn_g'], _D), rv(p['bn_b'], _D),
        rv(p['bn_rm'], _D), rv(p['bn_rv'], _D),
        wcast(p['oh1_w']), rv(p['oh1_b'], _BOT),
    ]

    in_specs = [pl.BlockSpec((1, _L, _D), lambda i: (i, 0, 0))]
    in_specs += [full_spec(a) for a in operands[1:]]

    out_shapes = [
        jax.ShapeDtypeStruct((bc, _D, _BOT), f32),
        jax.ShapeDtypeStruct((bc, _L, 1), f32),
    ]
    out_specs = [
        pl.BlockSpec((1, _D, _BOT), lambda i: (i, 0, 0)),
        pl.BlockSpec((1, _L, 1), lambda i: (i, 0, 0)),
    ]

    t_all, pb_all = pl.pallas_call(
        _row_kernel,
        grid=(bc,),
        in_specs=in_specs,
        out_specs=out_specs,
        out_shape=out_shapes,
        compiler_params=pltpu.CompilerParams(
            dimension_semantics=("parallel",)),
    )(*operands)

    t_flat = t_all.reshape(bc, _D * _BOT)

    dec_all = pl.pallas_call(
        _head_kernel,
        out_shape=jax.ShapeDtypeStruct((bc, _PRED), f32),
    )(t_flat, p['oh2_w'].astype(bf16), rv(p['oh2_b'], _PRED), mu, sd)

    dec_out = jnp.transpose(dec_all.reshape(_B, _C, _PRED), (0, 2, 1))

    pb_bc = pb_all.reshape(_B, _C, _L)
    bmask = pb_bc >= 0.5
    bmask = bmask.at[:, :, 0].set(True)
    boundary_prob = jnp.stack([1.0 - pb_bc, pb_bc], axis=-1)
    selected = jnp.where(bmask, pb_bc, 1.0 - pb_bc)[..., None]

    return dec_out, bmask, boundary_prob, selected


# speedup vs baseline: 1.1531x; 1.1531x over previous
import numpy as np
import jax
import jax.numpy as jnp
from jax.experimental import pallas as pl
from jax.experimental.pallas import tpu as pltpu

_B, _L, _C, _D, _PRED, _NH, _DFF, _NL, _BOT = 4, 512, 7, 128, 96, 8, 256, 2, 16
_DH = _D // _NH

f32 = jnp.float32
bf16 = jnp.bfloat16


def _posemb_np():
    pos = np.arange(_L)[:, None].astype(np.float32)
    div = np.exp(np.arange(0, _D, 2).astype(np.float32) * (-np.log(10000.0) / _D))
    pe = np.zeros((_L, _D), np.float32)
    pe[:, 0::2] = np.sin(pos * div)
    pe[:, 1::2] = np.cos(pos * div)
    return pe


_PE = _posemb_np()


def _ln(x, g, b):
    m = jnp.mean(x, axis=1, keepdims=True)
    v = jnp.mean((x - m) ** 2, axis=1, keepdims=True)
    return (x - m) / jnp.sqrt(v + 1e-5) * g + b


def _bdot(a, b):
    return jax.lax.dot_general(a.astype(bf16), b.astype(bf16),
                               (((1,), (0,)), ((), ())),
                               preferred_element_type=f32)


def _dot(a, b):
    return jax.lax.dot_general(a, b, (((1,), (0,)), ((), ())),
                               preferred_element_type=f32)


def _dot_t(a, b):
    return jax.lax.dot_general(a, b, (((1,), (1,)), ((), ())),
                               preferred_element_type=f32)


def _embed_kernel(xr_ref, ve1w_ref, ve1b_ref, ve2w_ref, ve2b_ref,
                  h_ref, mu_ref, sd_ref):
    x = xr_ref[...]
    m = jnp.mean(x, axis=1, keepdims=True)
    xc = x - m
    m2 = jnp.mean(xc, axis=1, keepdims=True)
    var = jnp.mean((xc - m2) ** 2, axis=1, keepdims=True)
    std = jnp.sqrt(var + 1e-5)
    xn = xc / std
    v = _bdot(xn, ve1w_ref[...]) + ve1b_ref[...]
    v16 = v.astype(bf16).astype(f32)
    h_ref[...] = _bdot(v16, ve2w_ref[...]) + ve2b_ref[...]
    mu_ref[...] = m
    sd_ref[...] = std


def _row_kernel(*refs):
    (h_ref, rpw_ref, rpb_ref, pe_ref) = refs[:4]
    layer_refs = refs[4:4 + 16 * _NL]
    (bng_ref, bnb_ref, bnrm_ref, bnrv_ref, oh1w_ref, oh1b_ref) = refs[4 + 16 * _NL:4 + 16 * _NL + 6]
    (t_ref, pb_ref) = refs[4 + 16 * _NL + 6:]

    h = h_ref[0]

    resid = _dot(h, rpw_ref[...]) + rpb_ref[...]

    hq = h.astype(bf16).astype(f32)
    nrm = jnp.sqrt(jnp.sum(hq * hq, axis=1, keepdims=True))
    dots = jnp.sum(hq[:_L - 1] * hq[1:], axis=1, keepdims=True)
    cos = dots / (nrm[:_L - 1] * nrm[1:] + 1e-12)
    pbt = jnp.clip((1.0 - cos) * 0.5, 0.0, 1.0)
    pb = jnp.concatenate([jnp.ones((1, 1), f32), pbt], axis=0)

    isub = jax.lax.broadcasted_iota(jnp.int32, (_L, 1), 0)
    bmaskf = jnp.where(pb >= 0.5, 1.0, 0.0)
    bmaskf = jnp.where(isub == 0, 1.0, bmaskf)

    ii = jax.lax.broadcasted_iota(jnp.int32, (_L, _L), 0).astype(f32)
    jj = jax.lax.broadcasted_iota(jnp.int32, (_L, _L), 1).astype(f32)
    tri = jnp.where(jj <= ii, 1.0, 0.0)
    c_s = _dot(tri, bmaskf)

    eye = jnp.where(ii == jj, 1.0, 0.0)
    ones_row = jnp.ones((1, _L), f32)
    b_l = _dot(ones_row, eye * bmaskf)
    c_l = _dot(b_l, jnp.where(ii <= jj, 1.0, 0.0))

    P = jnp.where((jnp.abs(c_l - 1.0 - ii) < 0.5) & (b_l > 0.5), 1.0, 0.0)
    G = jnp.where(jnp.abs(c_s - 1.0 - jj) < 0.5, 1.0, 0.0)

    z = _dot(P, h) + pe_ref[...]

    inv_sqrt_dh = 1.0 / np.sqrt(float(_DH))
    for l in range(_NL):
        (wq, bq, wk, bk, wv, bv, wo, bo, c1w, c1b, c2w, c2b,
         g1, b1, g2, b2) = (r[...] for r in layer_refs[16 * l:16 * (l + 1)])
        q = (_dot(z, wq) + bq) * inv_sqrt_dh
        kk = _dot(z, wk) + bk
        vv = _dot(z, wv) + bv
        outs = []
        for hh in range(_NH):
            sl = slice(hh * _DH, (hh + 1) * _DH)
            s = _dot_t(q[:, sl], kk[:, sl])
            s = s - jnp.max(s, axis=1, keepdims=True)
            e = jnp.exp(s)
            r = 1.0 / jnp.sum(e, axis=1, keepdims=True)
            outs.append(_dot(e, vv[:, sl]) * r)
        o = jnp.concatenate(outs, axis=1)
        o = _dot(o, wo) + bo
        x1 = _ln(z + o, g1, b1)
        y = jax.nn.gelu(_dot(x1, c1w) + c1b)
        y = _dot(y, c2w) + c2b
        z = _ln(x1 + y, g2, b2)

    z = (z - bnrm_ref[...]) / jnp.sqrt(bnrv_ref[...] + 1e-5) * bng_ref[...] + bnb_ref[...]

    expanded = _dot(G, z)

    w = jnp.clip(pb, 1e-4, 1.0)
    bb = w * expanded
    aa = 1.0 - w
    d = 1
    while d < _L:
        a_sh = jnp.concatenate([jnp.ones((d, 1), f32), aa[:_L - d]], axis=0)
        b_sh = jnp.concatenate([jnp.zeros((d, _D), f32), bb[:_L - d]], axis=0)
        bb = aa * b_sh + bb
        aa = aa * a_sh
        d *= 2
    hs = bb + resid

    t = _dot(jnp.transpose(hs), oh1w_ref[...]) + oh1b_ref[...]

    t_ref[...] = t.reshape(1, _D, _BOT)
    pb_ref[...] = pb.reshape(1, _L, 1)


def _head_kernel(t_ref, w_ref, b_ref, mu_ref, sd_ref, out_ref):
    t = t_ref[...]
    dec = _dot(t, w_ref[...]) + b_ref[...]
    out_ref[...] = dec * sd_ref[...] + mu_ref[...]


def kernel(x_enc, x_mark_enc, x_dec, x_mark_dec, params):
    p = params
    bc = _B * _C

    xr = jnp.transpose(x_enc, (0, 2, 1)).reshape(bc, _L).astype(f32)

    def rv(a, n):
        return a.reshape(1, n)

    def full_spec(a):
        nd = a.ndim
        return pl.BlockSpec(a.shape, lambda i, _n=nd: (0,) * _n)

    emb_operands = [xr, p['ve1_w'], rv(p['ve1_b'], _BOT),
                    p['ve2_w'], rv(p['ve2_b'], _L * _D)]
    h_all, mu, sd = pl.pallas_call(
        _embed_kernel,
        out_shape=[jax.ShapeDtypeStruct((bc, _L * _D), f32),
                   jax.ShapeDtypeStruct((bc, 1), f32),
                   jax.ShapeDtypeStruct((bc, 1), f32)],
    )(*emb_operands)

    operands = [
        h_all.reshape(bc, _L, _D),
        (p['rp_w']), rv(p['rp_b'], _D),
        jnp.asarray(_PE),
    ]
    for lp in p['layers']:
        operands += [
            (lp['wq']), rv(lp['bq'], _D), (lp['wk']), rv(lp['bk'], _D),
            (lp['wv']), rv(lp['bv'], _D), (lp['wo']), rv(lp['bo'], _D),
            (lp['c1_w']), rv(lp['c1_b'], _DFF), (lp['c2_w']), rv(lp['c2_b'], _D),
            rv(lp['ln1_g'], _D), rv(lp['ln1_b'], _D),
            rv(lp['ln2_g'], _D), rv(lp['ln2_b'], _D),
        ]
    operands += [
        rv(p['bn_g'], _D), rv(p['bn_b'], _D),
        rv(p['bn_rm'], _D), rv(p['bn_rv'], _D),
        (p['oh1_w']), rv(p['oh1_b'], _BOT),
    ]

    in_specs = [pl.BlockSpec((1, _L, _D), lambda i: (i, 0, 0))]
    in_specs += [full_spec(a) for a in operands[1:]]

    out_shapes = [
        jax.ShapeDtypeStruct((bc, _D, _BOT), f32),
        jax.ShapeDtypeStruct((bc, _L, 1), f32),
    ]
    out_specs = [
        pl.BlockSpec((1, _D, _BOT), lambda i: (i, 0, 0)),
        pl.BlockSpec((1, _L, 1), lambda i: (i, 0, 0)),
    ]

    t_all, pb_all = pl.pallas_call(
        _row_kernel,
        grid=(bc,),
        in_specs=in_specs,
        out_specs=out_specs,
        out_shape=out_shapes,
        compiler_params=pltpu.CompilerParams(
            dimension_semantics=("parallel",)),
    )(*operands)

    t_flat = t_all.reshape(bc, _D * _BOT)

    dec_all = pl.pallas_call(
        _head_kernel,
        out_shape=jax.ShapeDtypeStruct((bc, _PRED), f32),
    )(t_flat, p['oh2_w'], rv(p['oh2_b'], _PRED), mu, sd)

    dec_out = jnp.transpose(dec_all.reshape(_B, _C, _PRED), (0, 2, 1))

    pb_bc = pb_all.reshape(_B, _C, _L)
    bmask = pb_bc >= 0.5
    bmask = bmask.at[:, :, 0].set(True)
    boundary_prob = jnp.stack([1.0 - pb_bc, pb_bc], axis=-1)
    selected = jnp.where(bmask, pb_bc, 1.0 - pb_bc)[..., None]

    return dec_out, bmask, boundary_prob, selected


# speedup vs baseline: 1.4522x; 1.2594x over previous
import numpy as np
import jax
import jax.numpy as jnp
from jax.experimental import pallas as pl
from jax.experimental.pallas import tpu as pltpu

_B, _L, _C, _D, _PRED, _NH, _DFF, _NL, _BOT = 4, 512, 7, 128, 96, 8, 256, 2, 16
_DH = _D // _NH

f32 = jnp.float32
bf16 = jnp.bfloat16


def _posemb_np():
    pos = np.arange(_L)[:, None].astype(np.float32)
    div = np.exp(np.arange(0, _D, 2).astype(np.float32) * (-np.log(10000.0) / _D))
    pe = np.zeros((_L, _D), np.float32)
    pe[:, 0::2] = np.sin(pos * div)
    pe[:, 1::2] = np.cos(pos * div)
    return pe


_PE = _posemb_np()


def _ln(x, g, b):
    m = jnp.mean(x, axis=1, keepdims=True)
    v = jnp.mean((x - m) ** 2, axis=1, keepdims=True)
    return (x - m) * jax.lax.rsqrt(v + 1e-5) * g + b


def _bdot(a, b):
    return jax.lax.dot_general(a.astype(bf16), b.astype(bf16),
                               (((1,), (0,)), ((), ())),
                               preferred_element_type=f32)


def _dot(a, b):
    return jax.lax.dot_general(a, b, (((1,), (0,)), ((), ())),
                               preferred_element_type=f32)


def _dot_t(a, b):
    return jax.lax.dot_general(a, b, (((1,), (1,)), ((), ())),
                               preferred_element_type=f32)


def _embed_kernel(xr_ref, ve1w_ref, ve1b_ref, ve2w_ref, ve2b_ref,
                  h_ref, mu_ref, sd_ref):
    x = xr_ref[...]
    m = jnp.mean(x, axis=1, keepdims=True)
    xc = x - m
    m2 = jnp.mean(xc, axis=1, keepdims=True)
    var = jnp.mean((xc - m2) ** 2, axis=1, keepdims=True)
    std = jnp.sqrt(var + 1e-5)
    xn = xc / std
    v = _bdot(xn, ve1w_ref[...]) + ve1b_ref[...]
    v16 = v.astype(bf16).astype(f32)
    h_ref[...] = _bdot(v16, ve2w_ref[...]) + ve2b_ref[...]
    mu_ref[...] = m
    sd_ref[...] = std


_RPB = 1


def _row_kernel(*refs):
    (h_ref, rpw_ref, rpb_ref, pe_ref) = refs[:4]
    layer_refs = refs[4:4 + 16 * _NL]
    (bng_ref, bnb_ref, bnrm_ref, bnrv_ref, oh1w_ref, oh1b_ref) = refs[4 + 16 * _NL:4 + 16 * _NL + 6]
    (t_ref, pb_ref) = refs[4 + 16 * _NL + 6:]

    for _r in range(_RPB):
        _one_row(h_ref[_r], rpw_ref, rpb_ref, pe_ref, layer_refs,
                 bng_ref, bnb_ref, bnrm_ref, bnrv_ref, oh1w_ref, oh1b_ref,
                 t_ref, pb_ref, _r)


def _one_row(h, rpw_ref, rpb_ref, pe_ref, layer_refs,
             bng_ref, bnb_ref, bnrm_ref, bnrv_ref, oh1w_ref, oh1b_ref,
             t_ref, pb_ref, _r):

    resid = _dot(h, rpw_ref[...]) + rpb_ref[...]

    hq = h.astype(bf16).astype(f32)
    nrm = jnp.sqrt(jnp.sum(hq * hq, axis=1, keepdims=True))
    dots = jnp.sum(hq[:_L - 1] * hq[1:], axis=1, keepdims=True)
    cos = dots / (nrm[:_L - 1] * nrm[1:] + 1e-12)
    pbt = jnp.clip((1.0 - cos) * 0.5, 0.0, 1.0)
    pb = jnp.concatenate([jnp.ones((1, 1), f32), pbt], axis=0)

    isub = jax.lax.broadcasted_iota(jnp.int32, (_L, 1), 0)
    bmaskf = jnp.where(pb >= 0.5, 1.0, 0.0)
    bmaskf = jnp.where(isub == 0, 1.0, bmaskf)

    ii = jax.lax.broadcasted_iota(jnp.int32, (_L, _L), 0).astype(f32)
    jj = jax.lax.broadcasted_iota(jnp.int32, (_L, _L), 1).astype(f32)
    tri = jnp.where(jj <= ii, 1.0, 0.0)
    c_s = _dot(tri, bmaskf)

    eye = jnp.where(ii == jj, 1.0, 0.0)
    ones_row = jnp.ones((1, _L), f32)
    b_l = _dot(ones_row, eye * bmaskf)
    c_l = _dot(b_l, jnp.where(ii <= jj, 1.0, 0.0))

    P = jnp.where((jnp.abs(c_l - 1.0 - ii) < 0.5) & (b_l > 0.5), 1.0, 0.0)
    G = jnp.where(jnp.abs(c_s - 1.0 - jj) < 0.5, 1.0, 0.0)

    z = _dot(P, h) + pe_ref[...]

    inv_sqrt_dh = 1.0 / np.sqrt(float(_DH))
    for l in range(_NL):
        (wq, bq, wk, bk, wv, bv, wo, bo, c1w, c1b, c2w, c2b,
         g1, b1, g2, b2) = (r[...] for r in layer_refs[16 * l:16 * (l + 1)])
        q = (_dot(z, wq) + bq) * (inv_sqrt_dh * np.log2(np.e))
        kk = _dot(z, wk) + bk
        vv = _dot(z, wv) + bv
        outs = []
        for hh in range(_NH):
            sl = slice(hh * _DH, (hh + 1) * _DH)
            s = _dot_t(q[:, sl], kk[:, sl])
            s = s - jnp.max(s, axis=1, keepdims=True)
            e = jnp.exp2(s)
            a = e / jnp.sum(e, axis=1, keepdims=True)
            outs.append(_dot(a, vv[:, sl]))
        o = jnp.concatenate(outs, axis=1)
        o = _dot(o, wo) + bo
        x1 = _ln(z + o, g1, b1)
        y = jax.nn.gelu(_dot(x1, c1w) + c1b)
        y = _dot(y, c2w) + c2b
        z = _ln(x1 + y, g2, b2)

    z = (z - bnrm_ref[...]) / jnp.sqrt(bnrv_ref[...] + 1e-5) * bng_ref[...] + bnb_ref[...]

    expanded = _dot(G, z)

    w = jnp.clip(pb, 1e-4, 1.0)
    bb = w * expanded
    aa = 1.0 - w
    d = 1
    while d < _L:
        a_sh = jnp.concatenate([jnp.ones((d, 1), f32), aa[:_L - d]], axis=0)
        b_sh = jnp.concatenate([jnp.zeros((d, _D), f32), bb[:_L - d]], axis=0)
        bb = aa * b_sh + bb
        aa = aa * a_sh
        d *= 2
    hs = bb + resid

    t = _dot(jnp.transpose(hs), oh1w_ref[...]) + oh1b_ref[...]

    t_ref[_r, :, :] = t
    pb_ref[_r, :, :] = pb


def _head_kernel(t_ref, w_ref, b_ref, mu_ref, sd_ref, out_ref):
    t = t_ref[...]
    dec = _dot(t, w_ref[...]) + b_ref[...]
    out_ref[...] = dec * sd_ref[...] + mu_ref[...]


def kernel(x_enc, x_mark_enc, x_dec, x_mark_dec, params):
    p = params
    bc = _B * _C

    xr = jnp.transpose(x_enc, (0, 2, 1)).reshape(bc, _L).astype(f32)

    def rv(a, n):
        return a.reshape(1, n)

    def full_spec(a):
        nd = a.ndim
        return pl.BlockSpec(a.shape, lambda i, _n=nd: (0,) * _n)

    emb_operands = [xr, p['ve1_w'], rv(p['ve1_b'], _BOT),
                    p['ve2_w'], rv(p['ve2_b'], _L * _D)]
    h_all, mu, sd = pl.pallas_call(
        _embed_kernel,
        out_shape=[jax.ShapeDtypeStruct((bc, _L * _D), f32),
                   jax.ShapeDtypeStruct((bc, 1), f32),
                   jax.ShapeDtypeStruct((bc, 1), f32)],
    )(*emb_operands)

    operands = [
        h_all.reshape(bc, _L, _D),
        (p['rp_w']), rv(p['rp_b'], _D),
        jnp.asarray(_PE),
    ]
    for lp in p['layers']:
        operands += [
            (lp['wq']), rv(lp['bq'], _D), (lp['wk']), rv(lp['bk'], _D),
            (lp['wv']), rv(lp['bv'], _D), (lp['wo']), rv(lp['bo'], _D),
            (lp['c1_w']), rv(lp['c1_b'], _DFF), (lp['c2_w']), rv(lp['c2_b'], _D),
            rv(lp['ln1_g'], _D), rv(lp['ln1_b'], _D),
            rv(lp['ln2_g'], _D), rv(lp['ln2_b'], _D),
        ]
    operands += [
        rv(p['bn_g'], _D), rv(p['bn_b'], _D),
        rv(p['bn_rm'], _D), rv(p['bn_rv'], _D),
        (p['oh1_w']), rv(p['oh1_b'], _BOT),
    ]

    in_specs = [pl.BlockSpec((_RPB, _L, _D), lambda i: (i, 0, 0))]
    in_specs += [full_spec(a) for a in operands[1:]]

    out_shapes = [
        jax.ShapeDtypeStruct((bc, _D, _BOT), f32),
        jax.ShapeDtypeStruct((bc, _L, 1), f32),
    ]
    out_specs = [
        pl.BlockSpec((_RPB, _D, _BOT), lambda i: (i, 0, 0)),
        pl.BlockSpec((_RPB, _L, 1), lambda i: (i, 0, 0)),
    ]

    t_all, pb_all = pl.pallas_call(
        _row_kernel,
        grid=(bc // _RPB,),
        in_specs=in_specs,
        out_specs=out_specs,
        out_shape=out_shapes,
        compiler_params=pltpu.CompilerParams(
            dimension_semantics=("parallel",)),
    )(*operands)

    t_flat = t_all.reshape(bc, _D * _BOT)

    dec_all = pl.pallas_call(
        _head_kernel,
        out_shape=jax.ShapeDtypeStruct((bc, _PRED), f32),
    )(t_flat, p['oh2_w'], rv(p['oh2_b'], _PRED), mu, sd)

    dec_out = jnp.transpose(dec_all.reshape(_B, _C, _PRED), (0, 2, 1))

    pb_bc = pb_all.reshape(_B, _C, _L)
    bmask = pb_bc >= 0.5
    bmask = bmask.at[:, :, 0].set(True)
    boundary_prob = jnp.stack([1.0 - pb_bc, pb_bc], axis=-1)
    selected = jnp.where(bmask, pb_bc, 1.0 - pb_bc)[..., None]

    return dec_out, bmask, boundary_prob, selected
